# baseline (speedup 1.0000x reference)
import functools

import jax
import jax.numpy as jnp
from jax import lax
from jax.experimental import pallas as pl
from jax.experimental.pallas import tpu as pltpu
from jax.experimental.pallas import tpu_sc as plsc

_N = 10240
_E = 76800
_B = 16
_L = 640
_D = 1024
_H = 256
_R = 3
_HH = 128
_NSUB = 16
_EPT = _E // _NSUB
_CH = 48
_NCH = _EPT // _CH
_NB = 256
_NPT = _N // 16


def _mm_body(x_ref, w_ref, wg_ref, t2_ref):
    xb = x_ref[...]
    g = jax.nn.sigmoid(
        jnp.dot(xb, wg_ref[...], preferred_element_type=jnp.float32))
    for r in range(_R):
        t = jnp.dot(xb, w_ref[r], preferred_element_type=jnp.float32)
        scale = g[:, r][:, None]
        t2_ref[0, r] = t[:, :_HH] * scale
        t2_ref[1, r] = t[:, _HH:] * scale


def _build_mm(interpret=False):
    return pl.pallas_call(
        _mm_body,
        grid=(_N // _NB,),
        in_specs=[
            pl.BlockSpec((_NB, _D), lambda n: (n, 0)),
            pl.BlockSpec((_R, _D, _H), lambda n: (0, 0, 0)),
            pl.BlockSpec((_D, _HH), lambda n: (0, 0)),
        ],
        out_specs=pl.BlockSpec((2, _R, _NB, _HH), lambda n: (0, 0, n, 0)),
        out_shape=jax.ShapeDtypeStruct((2, _R, _N, _HH), jnp.float32),
        interpret=interpret,
    )


def _build_sc():
    mesh = plsc.VectorSubcoreMesh(core_axis_name="c", subcore_axis_name="s")

    @functools.partial(
        pl.kernel,
        mesh=mesh,
        out_type=jax.ShapeDtypeStruct((2, _N, _HH), jnp.float32),
        compiler_params=pltpu.CompilerParams(needs_layout_passes=False),
        scratch_types=[
            pltpu.VMEM((_NCH, _CH), jnp.int32),
            pltpu.VMEM((_NCH, _CH), jnp.int32),
            pltpu.VMEM((_EPT,), jnp.float32),
            pltpu.VMEM((2 * _CH, _HH), jnp.float32),
            pltpu.VMEM_SHARED((_N, _HH), jnp.float32),
            pltpu.SemaphoreType.DMA,
            pltpu.SemaphoreType.DMA,
            pltpu.SemaphoreType.DMA,
            pltpu.SemaphoreType.DMA,
        ],
    )
    def sc(t2, gidx_in, dst3, nrm, out,
           gidx, didx, nrmv, rowsb, hsh, rsem0, rsem1, ssem0, ssem1):
        c = lax.axis_index("c")
        s = lax.axis_index("s")
        nbase = s * _NPT
        rsems = (rsem0, rsem1)
        ssems = (ssem0, ssem1)
        slots = [rowsb.at[pl.ds(b * _CH, _CH)] for b in range(2)]

        pltpu.sync_copy(gidx_in.at[s], gidx)
        pltpu.sync_copy(dst3.at[s], didx)
        pltpu.sync_copy(nrm.at[s], nrmv)

        cv = jnp.full((16,), c * (_R * _N), jnp.int32)

        def addc(j, carry):
            for t in range(_CH // 16):
                sl = pl.ds(t * 16, 16)
                gidx[j, sl] = gidx[j, sl] + cv
            return carry
        lax.fori_loop(0, _NCH, addc, 0)

        z16 = jnp.zeros((16,), jnp.float32)

        def zrow(r, carry):
            for m in range(_HH // 16):
                rowsb[r, pl.ds(m * 16, 16)] = z16
            return carry
        lax.fori_loop(0, 64, zrow, 0)
        for q in range(_NPT // 64):
            pltpu.sync_copy(rowsb.at[pl.ds(0, 64)],
                            hsh.at[pl.ds(nbase + q * 64, 64)])

        plsc.subcore_barrier()

        pltpu.async_copy(t2.at[gidx.at[0]], slots[0], rsems[0])

        def step(ii, carry):
            for b in range(2):
                j = ii * 2 + b
                rb = slots[b]
                pltpu.make_async_copy(t2.at[gidx.at[j]], rb,
                                      rsems[b]).wait()

                @pl.when(j + 1 < _NCH)
                def _():
                    @pl.when(j >= 1)
                    def _():
                        pltpu.make_async_copy(slots[1 - b],
                                              hsh.at[didx.at[j]],
                                              ssems[1 - b]).wait()
                    pltpu.async_copy(t2.at[gidx.at[j + 1]], slots[1 - b],
                                     rsems[1 - b])

                base = j * _CH

                def edge(kk, kc):
                    k = kk * 4
                    cv0 = plsc.load_gather(
                        nrmv, [jnp.full((16,), base + k, jnp.int32)])
                    cv1 = plsc.load_gather(
                        nrmv, [jnp.full((16,), base + k + 1, jnp.int32)])
                    cv2 = plsc.load_gather(
                        nrmv, [jnp.full((16,), base + k + 2, jnp.int32)])
                    cv3 = plsc.load_gather(
                        nrmv, [jnp.full((16,), base + k + 3, jnp.int32)])
                    for m in range(_HH // 16):
                        sl = pl.ds(m * 16, 16)
                        rb[k, sl] = rb[k, sl] * cv0
                        rb[k + 1, sl] = rb[k + 1, sl] * cv1
                        rb[k + 2, sl] = rb[k + 2, sl] * cv2
                        rb[k + 3, sl] = rb[k + 3, sl] * cv3
                    return kc
                lax.fori_loop(0, _CH // 4, edge, 0)

                pltpu.async_copy(rb, hsh.at[didx.at[j]], ssems[b],
                                 add=True)
            return carry
        lax.fori_loop(0, _NCH // 2, step, 0)

        for b in range(2):
            pltpu.make_async_copy(slots[b], hsh.at[didx.at[0]],
                                  ssems[b]).wait()

        plsc.subcore_barrier()
        pltpu.sync_copy(hsh.at[pl.ds(nbase, _NPT)],
                        out.at[c, pl.ds(nbase, _NPT)])

    return sc


def _pool_body(h_ref, tok_ref, e1_ref, e2_ref, out_ref):
    g0 = jnp.maximum(h_ref[0, 0], 0.0)
    g1 = jnp.maximum(h_ref[1, 0], 0.0)
    tok = tok_ref[0]
    e1 = e1_ref[0]
    e2 = e2_ref[0]
    tokb = jnp.broadcast_to(tok, (_L, _HH))
    e1b = jnp.broadcast_to(e1, (_L, _HH))
    e2b = jnp.broadcast_to(e2, (_L, _HH))
    neg = jnp.float32(-1e12)
    d1 = jnp.float32(_L) - jnp.sum(e1)
    d2 = jnp.float32(_L) - jnp.sum(e2)
    out_ref[0, 0, pl.ds(0, _HH)] = jnp.max(
        jnp.where(tokb > 0.5, neg, g0), axis=0)
    out_ref[0, 0, pl.ds(_HH, _HH)] = jnp.max(
        jnp.where(tokb > 0.5, neg, g1), axis=0)
    out_ref[0, 0, pl.ds(2 * _HH, _HH)] = jnp.sum(
        jnp.where(e1b > 0.5, 0.0, g0), axis=0) / d1
    out_ref[0, 0, pl.ds(3 * _HH, _HH)] = jnp.sum(
        jnp.where(e1b > 0.5, 0.0, g1), axis=0) / d1
    out_ref[0, 0, pl.ds(4 * _HH, _HH)] = jnp.sum(
        jnp.where(e2b > 0.5, 0.0, g0), axis=0) / d2
    out_ref[0, 0, pl.ds(5 * _HH, _HH)] = jnp.sum(
        jnp.where(e2b > 0.5, 0.0, g1), axis=0) / d2


def _build_pool(interpret=False):
    return pl.pallas_call(
        _pool_body,
        grid=(_B,),
        in_specs=[
            pl.BlockSpec((2, 1, _L, _HH), lambda b: (0, b, 0, 0)),
            pl.BlockSpec((1, _L, 1), lambda b: (b, 0, 0)),
            pl.BlockSpec((1, _L, 1), lambda b: (b, 0, 0)),
            pl.BlockSpec((1, _L, 1), lambda b: (b, 0, 0)),
        ],
        out_specs=pl.BlockSpec((1, 1, 6 * _HH), lambda b: (b, 0, 0)),
        out_shape=jax.ShapeDtypeStruct((_B, 1, 6 * _HH), jnp.float32),
        interpret=interpret,
    )


def _head_body(gp_ref, be_ref, bnhg_ref, bnhb_ref, f1w_ref, f1b_ref,
               bn2g_ref, bn2b_ref, f2w_ref, f2b_ref, bn3g_ref, bn3b_ref,
               f3w_ref, f3b_ref, out_ref):
    def bn(v, g, b):
        m = jnp.mean(v, axis=0, keepdims=True)
        var = jnp.mean((v - m) * (v - m), axis=0, keepdims=True)
        return (v - m) / jnp.sqrt(var + 1e-5) * g + b

    be = bn(be_ref[...], bnhg_ref[...], bnhb_ref[...])
    be = jnp.maximum(
        jnp.dot(be, f1w_ref[...], preferred_element_type=jnp.float32)
        + f1b_ref[...], 0.0)
    emb = jnp.concatenate([gp_ref[...], be], axis=1)
    z = bn(emb, bn2g_ref[...], bn2b_ref[...])
    z = jnp.maximum(
        jnp.dot(z, f2w_ref[...], preferred_element_type=jnp.float32)
        + f2b_ref[...], 0.0)
    z = bn(z, bn3g_ref[...], bn3b_ref[...])
    out_ref[...] = (jnp.dot(z, f3w_ref[...], preferred_element_type=jnp.float32)
                    + f3b_ref[...])


def _build_head(interpret=False):
    return pl.pallas_call(
        _head_body,
        out_shape=jax.ShapeDtypeStruct((_B, _HH), jnp.float32),
        interpret=interpret,
    )


_mm_call = _build_mm()
_pool_call = _build_pool()
_head_call = _build_head()
_sc_cache = []


def _get_sc():
    if not _sc_cache:
        _sc_cache.append(_build_sc())
    return _sc_cache[0]


def kernel(x, edge_index, rel_type, norm, bert_embeddings, token_mask,
           event1_mask, event2_mask, W, gate_W, bnh_g, bnh_b, fc1_W, fc1_b,
           bn2_g, bn2_b, fc2_W, fc2_b, bn3_g, bn3_b, fc3_W, fc3_b):
    wg = jnp.zeros((_D, _HH), jnp.float32).at[:, :_R].set(gate_W[:, :, 0].T)
    t2 = _mm_call(x, W, wg)
    t2f = t2.reshape(2 * _R * _N, _HH)

    gidxb = (rel_type.astype(jnp.int32) * _N
             + edge_index[0].astype(jnp.int32)).reshape(_NSUB, _NCH, _CH)
    dst3 = edge_index[1].astype(jnp.int32).reshape(_NSUB, _NCH, _CH)
    nrm2 = norm[:, 0].reshape(_NSUB, _EPT)
    h2 = _get_sc()(t2f, gidxb, dst3, nrm2)

    h4 = h2.reshape(2, _B, _L, _HH)
    tokf = token_mask.astype(jnp.float32).reshape(_B, _L, 1)
    e1f = event1_mask.astype(jnp.float32).reshape(_B, _L, 1)
    e2f = event2_mask.astype(jnp.float32).reshape(_B, _L, 1)
    gp = _pool_call(h4, tokf, e1f, e2f).reshape(_B, 3 * _H)

    f3w = jnp.zeros((_H, _HH), jnp.float32).at[:, :6].set(fc3_W)
    f3b = jnp.zeros((1, _HH), jnp.float32).at[0, :6].set(fc3_b)
    out128 = _head_call(
        gp, bert_embeddings, bnh_g.reshape(1, -1), bnh_b.reshape(1, -1),
        fc1_W, fc1_b.reshape(1, -1), bn2_g.reshape(1, -1),
        bn2_b.reshape(1, -1), fc2_W, fc2_b.reshape(1, -1),
        bn3_g.reshape(1, -1), bn3_b.reshape(1, -1), f3w, f3b)
    return out128[:, :6]


# speedup vs baseline: 1.1576x; 1.1576x over previous
import functools

import jax
import jax.numpy as jnp
from jax import lax
from jax.experimental import pallas as pl
from jax.experimental.pallas import tpu as pltpu
from jax.experimental.pallas import tpu_sc as plsc

_N = 10240
_E = 76800
_B = 16
_L = 640
_D = 1024
_H = 256
_R = 3
_HH = 128
_NSUB = 16
_EPT = _E // _NSUB
_CH = 96
_NCH = _EPT // _CH
_NSLOT = 2
_NB = 256
_NPT = _N // 16


def _mm_body(x_ref, w_ref, wg_ref, t2_ref):
    xb = x_ref[...]
    g = jax.nn.sigmoid(
        jnp.dot(xb, wg_ref[...], preferred_element_type=jnp.float32))
    for r in range(_R):
        t = jnp.dot(xb, w_ref[r], preferred_element_type=jnp.float32)
        scale = g[:, r][:, None]
        t2_ref[0, r] = t[:, :_HH] * scale
        t2_ref[1, r] = t[:, _HH:] * scale


def _build_mm(interpret=False):
    return pl.pallas_call(
        _mm_body,
        grid=(_N // _NB,),
        in_specs=[
            pl.BlockSpec((_NB, _D), lambda n: (n, 0)),
            pl.BlockSpec((_R, _D, _H), lambda n: (0, 0, 0)),
            pl.BlockSpec((_D, _HH), lambda n: (0, 0)),
        ],
        out_specs=pl.BlockSpec((2, _R, _NB, _HH), lambda n: (0, 0, n, 0)),
        out_shape=jax.ShapeDtypeStruct((2, _R, _N, _HH), jnp.float32),
        interpret=interpret,
    )


def _build_sc():
    mesh = plsc.VectorSubcoreMesh(core_axis_name="c", subcore_axis_name="s")

    @functools.partial(
        pl.kernel,
        mesh=mesh,
        out_type=jax.ShapeDtypeStruct((2, _N, _HH), jnp.float32),
        compiler_params=pltpu.CompilerParams(needs_layout_passes=False),
        scratch_types=[
            pltpu.VMEM((_NCH, _CH), jnp.int32),
            pltpu.VMEM((_NCH, _CH), jnp.int32),
            pltpu.VMEM((_EPT,), jnp.float32),
            pltpu.VMEM((_NSLOT * _CH, _HH), jnp.float32),
            pltpu.VMEM_SHARED((_N, _HH), jnp.float32),
            pltpu.SemaphoreType.DMA,
            pltpu.SemaphoreType.DMA,
            pltpu.SemaphoreType.DMA,
            pltpu.SemaphoreType.DMA,
        ],
    )
    def sc(t2, gidx_in, dst3, nrm, out,
           gidx, didx, nrmv, rowsb, hsh, rsem0, rsem1, ssem0, ssem1):
        c = lax.axis_index("c")
        s = lax.axis_index("s")
        nbase = s * _NPT
        rsems = (rsem0, rsem1)
        ssems = (ssem0, ssem1)
        slots = [rowsb.at[pl.ds(b * _CH, _CH)] for b in range(_NSLOT)]

        pltpu.sync_copy(gidx_in.at[s], gidx)
        pltpu.sync_copy(dst3.at[s], didx)
        pltpu.sync_copy(nrm.at[s], nrmv)

        cv = jnp.full((16,), c * (_R * _N), jnp.int32)

        def addc(j, carry):
            for t in range(_CH // 16):
                sl = pl.ds(t * 16, 16)
                gidx[j, sl] = gidx[j, sl] + cv
            return carry
        lax.fori_loop(0, _NCH, addc, 0)

        z16 = jnp.zeros((16,), jnp.float32)

        def zrow(r, carry):
            for m in range(_HH // 16):
                rowsb[r, pl.ds(m * 16, 16)] = z16
            return carry
        lax.fori_loop(0, 64, zrow, 0)
        for q in range(_NPT // 64):
            pltpu.sync_copy(rowsb.at[pl.ds(0, 64)],
                            hsh.at[pl.ds(nbase + q * 64, 64)])

        plsc.subcore_barrier()

        pltpu.async_copy(t2.at[gidx.at[0]], slots[0], rsems[0])

        def step(ii, carry):
            for b in range(2):
                j = ii * 2 + b
                rb = slots[b]
                pltpu.make_async_copy(t2.at[gidx.at[j]], rb,
                                      rsems[b]).wait()

                @pl.when(j + 1 < _NCH)
                def _():
                    @pl.when(j >= 1)
                    def _():
                        pltpu.make_async_copy(slots[1 - b],
                                              hsh.at[didx.at[j]],
                                              ssems[1 - b]).wait()
                    pltpu.async_copy(t2.at[gidx.at[j + 1]], slots[1 - b],
                                     rsems[1 - b])

                base = j * _CH

                def edge(kk, kc):
                    k = kk * 2
                    cv0 = plsc.load_gather(
                        nrmv, [jnp.full((16,), base + k, jnp.int32)])
                    cv1 = plsc.load_gather(
                        nrmv, [jnp.full((16,), base + k + 1, jnp.int32)])
                    for m in range(_HH // 16):
                        sl = pl.ds(m * 16, 16)
                        rb[k, sl] = rb[k, sl] * cv0
                        rb[k + 1, sl] = rb[k + 1, sl] * cv1
                    return kc
                lax.fori_loop(0, _CH // 2, edge, 0)

                pltpu.async_copy(rb, hsh.at[didx.at[j]], ssems[b],
                                 add=True)
            return carry
        lax.fori_loop(0, _NCH // 2, step, 0)

        for b in range(2):
            pltpu.make_async_copy(slots[b], hsh.at[didx.at[0]],
                                  ssems[b]).wait()

        plsc.subcore_barrier()
        pltpu.sync_copy(hsh.at[pl.ds(nbase, _NPT)],
                        out.at[c, pl.ds(nbase, _NPT)])

    return sc


def _pool_body(h_ref, tok_ref, e1_ref, e2_ref, out_ref):
    g0 = jnp.maximum(h_ref[0, 0], 0.0)
    g1 = jnp.maximum(h_ref[1, 0], 0.0)
    tok = tok_ref[0]
    e1 = e1_ref[0]
    e2 = e2_ref[0]
    tokb = jnp.broadcast_to(tok, (_L, _HH))
    e1b = jnp.broadcast_to(e1, (_L, _HH))
    e2b = jnp.broadcast_to(e2, (_L, _HH))
    neg = jnp.float32(-1e12)
    d1 = jnp.float32(_L) - jnp.sum(e1)
    d2 = jnp.float32(_L) - jnp.sum(e2)
    out_ref[0, 0, pl.ds(0, _HH)] = jnp.max(
        jnp.where(tokb > 0.5, neg, g0), axis=0)
    out_ref[0, 0, pl.ds(_HH, _HH)] = jnp.max(
        jnp.where(tokb > 0.5, neg, g1), axis=0)
    out_ref[0, 0, pl.ds(2 * _HH, _HH)] = jnp.sum(
        jnp.where(e1b > 0.5, 0.0, g0), axis=0) / d1
    out_ref[0, 0, pl.ds(3 * _HH, _HH)] = jnp.sum(
        jnp.where(e1b > 0.5, 0.0, g1), axis=0) / d1
    out_ref[0, 0, pl.ds(4 * _HH, _HH)] = jnp.sum(
        jnp.where(e2b > 0.5, 0.0, g0), axis=0) / d2
    out_ref[0, 0, pl.ds(5 * _HH, _HH)] = jnp.sum(
        jnp.where(e2b > 0.5, 0.0, g1), axis=0) / d2


def _build_pool(interpret=False):
    return pl.pallas_call(
        _pool_body,
        grid=(_B,),
        in_specs=[
            pl.BlockSpec((2, 1, _L, _HH), lambda b: (0, b, 0, 0)),
            pl.BlockSpec((1, _L, 1), lambda b: (b, 0, 0)),
            pl.BlockSpec((1, _L, 1), lambda b: (b, 0, 0)),
            pl.BlockSpec((1, _L, 1), lambda b: (b, 0, 0)),
        ],
        out_specs=pl.BlockSpec((1, 1, 6 * _HH), lambda b: (b, 0, 0)),
        out_shape=jax.ShapeDtypeStruct((_B, 1, 6 * _HH), jnp.float32),
        interpret=interpret,
    )


def _head_body(gp_ref, be_ref, bnhg_ref, bnhb_ref, f1w_ref, f1b_ref,
               bn2g_ref, bn2b_ref, f2w_ref, f2b_ref, bn3g_ref, bn3b_ref,
               f3w_ref, f3b_ref, out_ref):
    def bn(v, g, b):
        m = jnp.mean(v, axis=0, keepdims=True)
        var = jnp.mean((v - m) * (v - m), axis=0, keepdims=True)
        return (v - m) / jnp.sqrt(var + 1e-5) * g + b

    be = bn(be_ref[...], bnhg_ref[...], bnhb_ref[...])
    be = jnp.maximum(
        jnp.dot(be, f1w_ref[...], preferred_element_type=jnp.float32)
        + f1b_ref[...], 0.0)
    emb = jnp.concatenate([gp_ref[...], be], axis=1)
    z = bn(emb, bn2g_ref[...], bn2b_ref[...])
    z = jnp.maximum(
        jnp.dot(z, f2w_ref[...], preferred_element_type=jnp.float32)
        + f2b_ref[...], 0.0)
    z = bn(z, bn3g_ref[...], bn3b_ref[...])
    out_ref[...] = (jnp.dot(z, f3w_ref[...], preferred_element_type=jnp.float32)
                    + f3b_ref[...])


def _build_head(interpret=False):
    return pl.pallas_call(
        _head_body,
        out_shape=jax.ShapeDtypeStruct((_B, _HH), jnp.float32),
        interpret=interpret,
    )


_mm_call = _build_mm()
_pool_call = _build_pool()
_head_call = _build_head()
_sc_cache = []


def _get_sc():
    if not _sc_cache:
        _sc_cache.append(_build_sc())
    return _sc_cache[0]


def kernel(x, edge_index, rel_type, norm, bert_embeddings, token_mask,
           event1_mask, event2_mask, W, gate_W, bnh_g, bnh_b, fc1_W, fc1_b,
           bn2_g, bn2_b, fc2_W, fc2_b, bn3_g, bn3_b, fc3_W, fc3_b):
    wg = jnp.zeros((_D, _HH), jnp.float32).at[:, :_R].set(gate_W[:, :, 0].T)
    t2 = _mm_call(x, W, wg)
    t2f = t2.reshape(2 * _R * _N, _HH)

    gidxb = (rel_type.astype(jnp.int32) * _N
             + edge_index[0].astype(jnp.int32)).reshape(_NSUB, _NCH, _CH)
    dst3 = edge_index[1].astype(jnp.int32).reshape(_NSUB, _NCH, _CH)
    nrm2 = norm[:, 0].reshape(_NSUB, _EPT)
    h2 = _get_sc()(t2f, gidxb, dst3, nrm2)

    h4 = h2.reshape(2, _B, _L, _HH)
    tokf = token_mask.astype(jnp.float32).reshape(_B, _L, 1)
    e1f = event1_mask.astype(jnp.float32).reshape(_B, _L, 1)
    e2f = event2_mask.astype(jnp.float32).reshape(_B, _L, 1)
    gp = _pool_call(h4, tokf, e1f, e2f).reshape(_B, 3 * _H)

    f3w = jnp.zeros((_H, _HH), jnp.float32).at[:, :6].set(fc3_W)
    f3b = jnp.zeros((1, _HH), jnp.float32).at[0, :6].set(fc3_b)
    out128 = _head_call(
        gp, bert_embeddings, bnh_g.reshape(1, -1), bnh_b.reshape(1, -1),
        fc1_W, fc1_b.reshape(1, -1), bn2_g.reshape(1, -1),
        bn2_b.reshape(1, -1), fc2_W, fc2_b.reshape(1, -1),
        bn3_g.reshape(1, -1), bn3_b.reshape(1, -1), f3w, f3b)
    return out128[:, :6]


# speedup vs baseline: 1.1643x; 1.0058x over previous
import functools

import jax
import jax.numpy as jnp
from jax import lax
from jax.experimental import pallas as pl
from jax.experimental.pallas import tpu as pltpu
from jax.experimental.pallas import tpu_sc as plsc

_N = 10240
_E = 76800
_B = 16
_L = 640
_D = 1024
_H = 256
_R = 3
_HH = 128
_NSUB = 16
_EPT = _E // _NSUB
_CH = 96
_NCH = _EPT // _CH
_NSLOT = 2
_NB = 256
_NPT = _N // 16


def _mm_body(x_ref, w_ref, wg_ref, t2_ref):
    xb = x_ref[...]
    g = jax.nn.sigmoid(
        jnp.dot(xb, wg_ref[...], preferred_element_type=jnp.float32))
    for r in range(_R):
        t = jnp.dot(xb, w_ref[r], preferred_element_type=jnp.float32)
        scale = g[:, r][:, None]
        t2_ref[0, r] = t[:, :_HH] * scale
        t2_ref[1, r] = t[:, _HH:] * scale


def _build_mm(interpret=False):
    return pl.pallas_call(
        _mm_body,
        grid=(_N // _NB,),
        in_specs=[
            pl.BlockSpec((_NB, _D), lambda n: (n, 0)),
            pl.BlockSpec((_R, _D, _H), lambda n: (0, 0, 0)),
            pl.BlockSpec((_D, _HH), lambda n: (0, 0)),
        ],
        out_specs=pl.BlockSpec((2, _R, _NB, _HH), lambda n: (0, 0, n, 0)),
        out_shape=jax.ShapeDtypeStruct((2, _R, _N, _HH), jnp.float32),
        interpret=interpret,
    )


def _build_sc():
    mesh = plsc.VectorSubcoreMesh(core_axis_name="c", subcore_axis_name="s")

    @functools.partial(
        pl.kernel,
        mesh=mesh,
        out_type=jax.ShapeDtypeStruct((2, _N, _HH), jnp.float32),
        compiler_params=pltpu.CompilerParams(needs_layout_passes=False),
        scratch_types=[
            pltpu.VMEM((_NCH, _CH), jnp.int32),
            pltpu.VMEM((_NCH, _CH), jnp.int32),
            pltpu.VMEM((_EPT,), jnp.float32),
            pltpu.VMEM((_NSLOT * _CH, _HH), jnp.float32),
            pltpu.VMEM_SHARED((_N, _HH), jnp.float32),
            pltpu.SemaphoreType.DMA,
            pltpu.SemaphoreType.DMA,
            pltpu.SemaphoreType.DMA,
            pltpu.SemaphoreType.DMA,
        ],
    )
    def sc(t2, gidx_in, dst3, nrm, out,
           gidx, didx, nrmv, rowsb, hsh, rsem0, rsem1, ssem0, ssem1):
        c = lax.axis_index("c")
        s = lax.axis_index("s")
        nbase = s * _NPT
        rsems = (rsem0, rsem1)
        ssems = (ssem0, ssem1)
        slots = [rowsb.at[pl.ds(b * _CH, _CH)] for b in range(_NSLOT)]

        pltpu.sync_copy(gidx_in.at[s], gidx)
        pltpu.sync_copy(dst3.at[s], didx)
        pltpu.sync_copy(nrm.at[s], nrmv)

        cv = jnp.full((16,), c * (_R * _N), jnp.int32)

        def addc(j, carry):
            for t in range(_CH // 16):
                sl = pl.ds(t * 16, 16)
                gidx[j, sl] = gidx[j, sl] + cv
            return carry
        lax.fori_loop(0, _NCH, addc, 0)

        z16 = jnp.zeros((16,), jnp.float32)

        def zrow(r, carry):
            for m in range(_HH // 16):
                rowsb[r, pl.ds(m * 16, 16)] = z16
            return carry
        lax.fori_loop(0, 64, zrow, 0)
        for q in range(_NPT // 64):
            pltpu.sync_copy(rowsb.at[pl.ds(0, 64)],
                            hsh.at[pl.ds(nbase + q * 64, 64)])

        plsc.subcore_barrier()

        pltpu.async_copy(t2.at[gidx.at[0]], slots[0], rsems[0])

        def step(ii, carry):
            for b in range(2):
                j = ii * 2 + b
                rb = slots[b]
                pltpu.make_async_copy(t2.at[gidx.at[j]], rb,
                                      rsems[b]).wait()

                @pl.when(j + 1 < _NCH)
                def _():
                    @pl.when(j >= 1)
                    def _():
                        pltpu.make_async_copy(slots[1 - b],
                                              hsh.at[didx.at[j]],
                                              ssems[1 - b]).wait()
                    pltpu.async_copy(t2.at[gidx.at[j + 1]], slots[1 - b],
                                     rsems[1 - b])

                base = j * _CH

                def edge(kk, kc):
                    k = kk * 4
                    cv0 = plsc.load_gather(
                        nrmv, [jnp.full((16,), base + k, jnp.int32)])
                    cv1 = plsc.load_gather(
                        nrmv, [jnp.full((16,), base + k + 1, jnp.int32)])
                    cv2 = plsc.load_gather(
                        nrmv, [jnp.full((16,), base + k + 2, jnp.int32)])
                    cv3 = plsc.load_gather(
                        nrmv, [jnp.full((16,), base + k + 3, jnp.int32)])
                    for m in range(_HH // 16):
                        sl = pl.ds(m * 16, 16)
                        rb[k, sl] = rb[k, sl] * cv0
                        rb[k + 1, sl] = rb[k + 1, sl] * cv1
                        rb[k + 2, sl] = rb[k + 2, sl] * cv2
                        rb[k + 3, sl] = rb[k + 3, sl] * cv3
                    return kc
                lax.fori_loop(0, _CH // 4, edge, 0)

                pltpu.async_copy(rb, hsh.at[didx.at[j]], ssems[b],
                                 add=True)
            return carry
        lax.fori_loop(0, _NCH // 2, step, 0)

        for b in range(2):
            pltpu.make_async_copy(slots[b], hsh.at[didx.at[0]],
                                  ssems[b]).wait()

        plsc.subcore_barrier()
        pltpu.sync_copy(hsh.at[pl.ds(nbase, _NPT)],
                        out.at[c, pl.ds(nbase, _NPT)])

    return sc


def _pool_body(h_ref, tok_ref, e1_ref, e2_ref, out_ref):
    g0 = jnp.maximum(h_ref[0, 0], 0.0)
    g1 = jnp.maximum(h_ref[1, 0], 0.0)
    tok = tok_ref[0]
    e1 = e1_ref[0]
    e2 = e2_ref[0]
    tokb = jnp.broadcast_to(tok, (_L, _HH))
    e1b = jnp.broadcast_to(e1, (_L, _HH))
    e2b = jnp.broadcast_to(e2, (_L, _HH))
    neg = jnp.float32(-1e12)
    d1 = jnp.float32(_L) - jnp.sum(e1)
    d2 = jnp.float32(_L) - jnp.sum(e2)
    out_ref[0, 0, pl.ds(0, _HH)] = jnp.max(
        jnp.where(tokb > 0.5, neg, g0), axis=0)
    out_ref[0, 0, pl.ds(_HH, _HH)] = jnp.max(
        jnp.where(tokb > 0.5, neg, g1), axis=0)
    out_ref[0, 0, pl.ds(2 * _HH, _HH)] = jnp.sum(
        jnp.where(e1b > 0.5, 0.0, g0), axis=0) / d1
    out_ref[0, 0, pl.ds(3 * _HH, _HH)] = jnp.sum(
        jnp.where(e1b > 0.5, 0.0, g1), axis=0) / d1
    out_ref[0, 0, pl.ds(4 * _HH, _HH)] = jnp.sum(
        jnp.where(e2b > 0.5, 0.0, g0), axis=0) / d2
    out_ref[0, 0, pl.ds(5 * _HH, _HH)] = jnp.sum(
        jnp.where(e2b > 0.5, 0.0, g1), axis=0) / d2


def _build_pool(interpret=False):
    return pl.pallas_call(
        _pool_body,
        grid=(_B,),
        in_specs=[
            pl.BlockSpec((2, 1, _L, _HH), lambda b: (0, b, 0, 0)),
            pl.BlockSpec((1, _L, 1), lambda b: (b, 0, 0)),
            pl.BlockSpec((1, _L, 1), lambda b: (b, 0, 0)),
            pl.BlockSpec((1, _L, 1), lambda b: (b, 0, 0)),
        ],
        out_specs=pl.BlockSpec((1, 1, 6 * _HH), lambda b: (b, 0, 0)),
        out_shape=jax.ShapeDtypeStruct((_B, 1, 6 * _HH), jnp.float32),
        interpret=interpret,
    )


def _head_body(gp_ref, be_ref, bnhg_ref, bnhb_ref, f1w_ref, f1b_ref,
               bn2g_ref, bn2b_ref, f2w_ref, f2b_ref, bn3g_ref, bn3b_ref,
               f3w_ref, f3b_ref, out_ref):
    def bn(v, g, b):
        m = jnp.mean(v, axis=0, keepdims=True)
        var = jnp.mean((v - m) * (v - m), axis=0, keepdims=True)
        return (v - m) / jnp.sqrt(var + 1e-5) * g + b

    be = bn(be_ref[...], bnhg_ref[...], bnhb_ref[...])
    be = jnp.maximum(
        jnp.dot(be, f1w_ref[...], preferred_element_type=jnp.float32)
        + f1b_ref[...], 0.0)
    emb = jnp.concatenate([gp_ref[...], be], axis=1)
    z = bn(emb, bn2g_ref[...], bn2b_ref[...])
    z = jnp.maximum(
        jnp.dot(z, f2w_ref[...], preferred_element_type=jnp.float32)
        + f2b_ref[...], 0.0)
    z = bn(z, bn3g_ref[...], bn3b_ref[...])
    out_ref[...] = (jnp.dot(z, f3w_ref[...], preferred_element_type=jnp.float32)
                    + f3b_ref[...])


def _build_head(interpret=False):
    return pl.pallas_call(
        _head_body,
        out_shape=jax.ShapeDtypeStruct((_B, _HH), jnp.float32),
        interpret=interpret,
    )


_mm_call = _build_mm()
_pool_call = _build_pool()
_head_call = _build_head()
_sc_cache = []


def _get_sc():
    if not _sc_cache:
        _sc_cache.append(_build_sc())
    return _sc_cache[0]


def kernel(x, edge_index, rel_type, norm, bert_embeddings, token_mask,
           event1_mask, event2_mask, W, gate_W, bnh_g, bnh_b, fc1_W, fc1_b,
           bn2_g, bn2_b, fc2_W, fc2_b, bn3_g, bn3_b, fc3_W, fc3_b):
    wg = jnp.zeros((_D, _HH), jnp.float32).at[:, :_R].set(gate_W[:, :, 0].T)
    t2 = _mm_call(x, W, wg)
    t2f = t2.reshape(2 * _R * _N, _HH)

    gidxb = (rel_type.astype(jnp.int32) * _N
             + edge_index[0].astype(jnp.int32)).reshape(_NSUB, _NCH, _CH)
    dst3 = edge_index[1].astype(jnp.int32).reshape(_NSUB, _NCH, _CH)
    nrm2 = norm[:, 0].reshape(_NSUB, _EPT)
    h2 = _get_sc()(t2f, gidxb, dst3, nrm2)

    h4 = h2.reshape(2, _B, _L, _HH)
    tokf = token_mask.astype(jnp.float32).reshape(_B, _L, 1)
    e1f = event1_mask.astype(jnp.float32).reshape(_B, _L, 1)
    e2f = event2_mask.astype(jnp.float32).reshape(_B, _L, 1)
    gp = _pool_call(h4, tokf, e1f, e2f).reshape(_B, 3 * _H)

    f3w = jnp.zeros((_H, _HH), jnp.float32).at[:, :6].set(fc3_W)
    f3b = jnp.zeros((1, _HH), jnp.float32).at[0, :6].set(fc3_b)
    out128 = _head_call(
        gp, bert_embeddings, bnh_g.reshape(1, -1), bnh_b.reshape(1, -1),
        fc1_W, fc1_b.reshape(1, -1), bn2_g.reshape(1, -1),
        bn2_b.reshape(1, -1), fc2_W, fc2_b.reshape(1, -1),
        bn3_g.reshape(1, -1), bn3_b.reshape(1, -1), f3w, f3b)
    return out128[:, :6]
